# baseline (speedup 1.0000x reference)
import jax
import jax.numpy as jnp
from jax.experimental import pallas as pl
from jax.experimental.pallas import tpu as pltpu

_NUM_ARCHS = 8
_R, _C = 2048, 4096
_BR = 128


def _copy_kernel(id_ref, w_ref, o_ref):
    o_ref[...] = w_ref[0]


def kernel(W, arch_id):
    idx = jnp.asarray(arch_id, jnp.int32).reshape((1,))
    return pl.pallas_call(
        _copy_kernel,
        grid_spec=pltpu.PrefetchScalarGridSpec(
            num_scalar_prefetch=1,
            grid=(_R // _BR,),
            in_specs=[
                pl.BlockSpec((1, _BR, _C), lambda i, id_ref: (id_ref[0], i, 0))
            ],
            out_specs=pl.BlockSpec((_BR, _C), lambda i, id_ref: (i, 0)),
        ),
        out_shape=jax.ShapeDtypeStruct((_R, _C), W.dtype),
    )(idx, W)


# speedup vs baseline: 1.1666x; 1.1666x over previous
import jax
import jax.numpy as jnp
from jax.experimental import pallas as pl
from jax.experimental.pallas import tpu as pltpu

_NUM_ARCHS = 8
_R, _C = 2048, 4096
_BR = 512


def _copy_kernel(id_ref, w_ref, o_ref):
    o_ref[...] = w_ref[0]


def kernel(W, arch_id):
    idx = jnp.asarray(arch_id, jnp.int32).reshape((1,))
    return pl.pallas_call(
        _copy_kernel,
        grid_spec=pltpu.PrefetchScalarGridSpec(
            num_scalar_prefetch=1,
            grid=(_R // _BR,),
            in_specs=[
                pl.BlockSpec((1, _BR, _C), lambda i, id_ref: (id_ref[0], i, 0))
            ],
            out_specs=pl.BlockSpec((_BR, _C), lambda i, id_ref: (i, 0)),
        ),
        out_shape=jax.ShapeDtypeStruct((_R, _C), W.dtype),
    )(idx, W)
